# Initial kernel scaffold; baseline (speedup 1.0000x reference)
import jax
import jax.numpy as jnp
from jax.experimental import pallas as pl


def kernel(boxes, scores):
    raise NotImplementedError("write your pallas kernel here")


# speedup vs baseline: 178.7051x; 178.7051x over previous
import functools

import jax
import jax.numpy as jnp
from jax import lax
from jax.experimental import pallas as pl
from jax.experimental.pallas import tpu as pltpu

_THRESH = 0.6


def _nms_chunk_kernel(nb, c,
                      x1r, y1r, x2r, y2r, ar,
                      x1c, y1c, x2c, y2c, ac,
                      sup0, supr):
    supr[...] = sup0[...]

    row_i = lax.broadcasted_iota(jnp.int32, (c, c), 0)
    col_i = lax.broadcasted_iota(jnp.int32, (c, c), 1)
    tri = (col_i > row_i)

    def iou_ge(cx1, cy1, cx2, cy2, ca, rx1, ry1, rx2, ry2, ra):
        xx1 = jnp.maximum(cx1, rx1)
        yy1 = jnp.maximum(cy1, ry1)
        xx2 = jnp.minimum(cx2, rx2)
        yy2 = jnp.minimum(cy2, ry2)
        w = jnp.maximum(0.0, xx2 - xx1 + 1.0)
        h = jnp.maximum(0.0, yy2 - yy1 + 1.0)
        inter = w * h
        union = ca + ra - inter
        return (inter / union) >= _THRESH

    def chunk_step(ci, _):
        base = ci * c
        cx1 = x1c[pl.ds(base, c), :]
        cy1 = y1c[pl.ds(base, c), :]
        cx2 = x2c[pl.ds(base, c), :]
        cy2 = y2c[pl.ds(base, c), :]
        ca = ac[pl.ds(base, c), :]

        rx1 = x1r[ci]
        ry1 = y1r[ci]
        rx2 = x2r[ci]
        ry2 = y2r[ci]
        ra = ar[ci]

        hit = iou_ge(cx1, cy1, cx2, cy2, ca, rx1, ry1, rx2, ry2, ra)
        m = jnp.where(hit & tri, 1.0, 0.0)

        s0 = supr[ci]

        def fix_cond(carry):
            _, changed = carry
            return changed

        def fix_body(carry):
            s, _ = carry
            active = 1.0 - s
            t = lax.dot_general(active, m, (((1,), (0,)), ((), ())),
                                preferred_element_type=jnp.float32)
            s_new = jnp.maximum(s0, jnp.where(t > 0.0, 1.0, 0.0))
            return s_new, jnp.any(s_new != s)

        s_fin, _ = lax.while_loop(fix_cond, fix_body, (s0, True))
        supr[ci] = s_fin
        keep_row = 1.0 - s_fin

        def cross(cj, _):
            ox1 = x1r[cj]
            oy1 = y1r[cj]
            ox2 = x2r[cj]
            oy2 = y2r[cj]
            oa = ar[cj]
            d = jnp.where(
                iou_ge(cx1, cy1, cx2, cy2, ca, ox1, oy1, ox2, oy2, oa),
                1.0, 0.0)
            t = lax.dot_general(keep_row, d, (((1,), (0,)), ((), ())),
                                preferred_element_type=jnp.float32)
            supr[cj] = jnp.maximum(supr[cj], jnp.where(t > 0.0, 1.0, 0.0))
            return 0

        lax.fori_loop(ci + 1, nb, cross, 0, unroll=False)
        return 0

    lax.fori_loop(0, nb, chunk_step, 0, unroll=False)


def _run_nms_sorted(x1s, y1s, x2s, y2s, areas_s, chunk):
    n = x1s.shape[0]
    c = chunk
    nb = -(-n // c)
    npad = nb * c
    pad = npad - n

    def prep(v):
        vp = jnp.pad(v, (0, pad))
        return vp.reshape(nb, 1, c), vp.reshape(npad, 1)

    (x1r, x1c) = prep(x1s)
    (y1r, y1c) = prep(y1s)
    (x2r, x2c) = prep(x2s)
    (y2r, y2c) = prep(y2s)
    (arr, arc) = prep(areas_s)
    sup0 = jnp.pad(jnp.zeros((n,), jnp.float32), (0, pad),
                   constant_values=1.0).reshape(nb, 1, c)

    sup = pl.pallas_call(
        functools.partial(_nms_chunk_kernel, nb, c),
        out_shape=jax.ShapeDtypeStruct((nb, 1, c), jnp.float32),
    )(x1r, y1r, x2r, y2r, arr, x1c, y1c, x2c, y2c, arc, sup0)
    return sup.reshape(npad)[:n]


def kernel(boxes, scores):
    n = boxes.shape[0]
    x1 = boxes[:, 0]
    y1 = boxes[:, 1]
    x2 = boxes[:, 2]
    y2 = boxes[:, 3]
    areas = (x2 - x1 + 1.0) * (y2 - y1 + 1.0)
    order = jnp.argsort(-scores)

    x1s = x1[order]
    y1s = y1[order]
    x2s = x2[order]
    y2s = y2[order]
    areas_s = areas[order]

    sup = _run_nms_sorted(x1s, y1s, x2s, y2s, areas_s, chunk=512)

    keep_sorted = sup < 0.5
    keep = jnp.zeros((n,), dtype=bool).at[order].set(keep_sorted)
    keep_f = keep.astype(boxes.dtype)
    labels = jnp.zeros((n, 1), dtype=boxes.dtype)
    bbox_pred = jnp.concatenate([labels, scores[:, None], boxes],
                                axis=1) * keep_f[:, None]
    bbox_num = jnp.sum(keep).astype(jnp.int32)[None]
    nms_keep_idx = jnp.nonzero(keep, size=n, fill_value=0)[0]
    return bbox_pred, bbox_num, nms_keep_idx


# speedup vs baseline: 200.3283x; 1.1210x over previous
import functools

import jax
import jax.numpy as jnp
from jax import lax
from jax.experimental import pallas as pl
from jax.experimental.pallas import tpu as pltpu

_THRESH = 0.6


def _nms_chunk_kernel(nb, c,
                      x1r, y1r, x2r, y2r, ar,
                      x1c, y1c, x2c, y2c, ac,
                      sup0, supr):
    supr[...] = sup0[...]

    row_i = lax.broadcasted_iota(jnp.int32, (c, c), 0)
    col_i = lax.broadcasted_iota(jnp.int32, (c, c), 1)
    tri = (col_i > row_i)

    def iou_ge(cx1, cy1, cx2, cy2, ca, rx1, ry1, rx2, ry2, ra):
        xx1 = jnp.maximum(cx1, rx1)
        yy1 = jnp.maximum(cy1, ry1)
        xx2 = jnp.minimum(cx2, rx2)
        yy2 = jnp.minimum(cy2, ry2)
        w = jnp.maximum(0.0, xx2 - xx1 + 1.0)
        h = jnp.maximum(0.0, yy2 - yy1 + 1.0)
        inter = w * h
        union = ca + ra - inter
        return (inter / union) >= _THRESH

    def chunk_step(ci, _):
        base = ci * c
        cx1 = x1c[pl.ds(base, c), :]
        cy1 = y1c[pl.ds(base, c), :]
        cx2 = x2c[pl.ds(base, c), :]
        cy2 = y2c[pl.ds(base, c), :]
        ca = ac[pl.ds(base, c), :]

        rx1 = x1r[ci]
        ry1 = y1r[ci]
        rx2 = x2r[ci]
        ry2 = y2r[ci]
        ra = ar[ci]

        hit = iou_ge(cx1, cy1, cx2, cy2, ca, rx1, ry1, rx2, ry2, ra)
        m = jnp.where(hit & tri, 1.0, 0.0)

        s0 = supr[ci]

        def fix_cond(carry):
            _, changed = carry
            return changed

        def fix_body(carry):
            s, _ = carry
            active = 1.0 - s
            t = lax.dot_general(active, m, (((1,), (0,)), ((), ())),
                                preferred_element_type=jnp.float32)
            s_new = jnp.maximum(s0, jnp.where(t > 0.0, 1.0, 0.0))
            return s_new, jnp.any(s_new != s)

        s_fin, _ = lax.while_loop(fix_cond, fix_body, (s0, True))
        supr[ci] = s_fin
        keep_row = 1.0 - s_fin

        def cross(cj, _):
            ox1 = x1r[cj]
            oy1 = y1r[cj]
            ox2 = x2r[cj]
            oy2 = y2r[cj]
            oa = ar[cj]
            d = jnp.where(
                iou_ge(cx1, cy1, cx2, cy2, ca, ox1, oy1, ox2, oy2, oa),
                1.0, 0.0)
            t = lax.dot_general(keep_row, d, (((1,), (0,)), ((), ())),
                                preferred_element_type=jnp.float32)
            supr[cj] = jnp.maximum(supr[cj], jnp.where(t > 0.0, 1.0, 0.0))
            return 0

        lax.fori_loop(ci + 1, nb, cross, 0, unroll=False)
        return 0

    lax.fori_loop(0, nb, chunk_step, 0, unroll=False)


def _run_nms_sorted(x1s, y1s, x2s, y2s, areas_s, chunk):
    n = x1s.shape[0]
    c = chunk
    nb = -(-n // c)
    npad = nb * c
    pad = npad - n

    def prep(v):
        vp = jnp.pad(v, (0, pad))
        return vp.reshape(nb, 1, c), vp.reshape(npad, 1)

    (x1r, x1c) = prep(x1s)
    (y1r, y1c) = prep(y1s)
    (x2r, x2c) = prep(x2s)
    (y2r, y2c) = prep(y2s)
    (arr, arc) = prep(areas_s)
    sup0 = jnp.pad(jnp.zeros((n,), jnp.float32), (0, pad),
                   constant_values=1.0).reshape(nb, 1, c)

    sup = pl.pallas_call(
        functools.partial(_nms_chunk_kernel, nb, c),
        out_shape=jax.ShapeDtypeStruct((nb, 1, c), jnp.float32),
    )(x1r, y1r, x2r, y2r, arr, x1c, y1c, x2c, y2c, arc, sup0)
    return sup.reshape(npad)[:n]


def kernel(boxes, scores):
    n = boxes.shape[0]
    x1 = boxes[:, 0]
    y1 = boxes[:, 1]
    x2 = boxes[:, 2]
    y2 = boxes[:, 3]
    areas = (x2 - x1 + 1.0) * (y2 - y1 + 1.0)
    order = jnp.argsort(-scores)

    x1s = x1[order]
    y1s = y1[order]
    x2s = x2[order]
    y2s = y2[order]
    areas_s = areas[order]

    sup = _run_nms_sorted(x1s, y1s, x2s, y2s, areas_s, chunk=1024)

    keep_sorted = sup < 0.5
    keep = jnp.zeros((n,), dtype=bool).at[order].set(keep_sorted)
    keep_f = keep.astype(boxes.dtype)
    labels = jnp.zeros((n, 1), dtype=boxes.dtype)
    bbox_pred = jnp.concatenate([labels, scores[:, None], boxes],
                                axis=1) * keep_f[:, None]
    bbox_num = jnp.sum(keep).astype(jnp.int32)[None]
    nms_keep_idx = jnp.nonzero(keep, size=n, fill_value=0)[0]
    return bbox_pred, bbox_num, nms_keep_idx


# speedup vs baseline: 202.6743x; 1.0117x over previous
import functools

import jax
import jax.numpy as jnp
from jax import lax
from jax.experimental import pallas as pl
from jax.experimental.pallas import tpu as pltpu

_THRESH = 0.6


def _nms_chunk_kernel(nb, c,
                      x1r, y1r, x2r, y2r, ar,
                      sup0, supr):
    supr[...] = sup0[...]

    row_i = lax.broadcasted_iota(jnp.int32, (c, c), 0)
    col_i = lax.broadcasted_iota(jnp.int32, (c, c), 1)
    tri = (col_i > row_i)

    def to_col(v):
        return jnp.transpose(v, (1, 0))

    def iou_ge(cx1, cy1, cx2, cy2, ca, rx1, ry1, rx2, ry2, ra):
        xx1 = jnp.maximum(cx1, rx1)
        yy1 = jnp.maximum(cy1, ry1)
        xx2 = jnp.minimum(cx2, rx2)
        yy2 = jnp.minimum(cy2, ry2)
        w = jnp.maximum(0.0, xx2 - xx1 + 1.0)
        h = jnp.maximum(0.0, yy2 - yy1 + 1.0)
        inter = w * h
        union = ca + ra - inter
        return (inter / union) >= _THRESH

    def chunk_step(ci, _):
        rx1 = x1r[ci]
        ry1 = y1r[ci]
        rx2 = x2r[ci]
        ry2 = y2r[ci]
        ra = ar[ci]

        cx1 = to_col(rx1)
        cy1 = to_col(ry1)
        cx2 = to_col(rx2)
        cy2 = to_col(ry2)
        ca = to_col(ra)

        hit = iou_ge(cx1, cy1, cx2, cy2, ca, rx1, ry1, rx2, ry2, ra)
        m = jnp.where(hit & tri, 1.0, 0.0)

        s0 = supr[ci]

        def fix_cond(carry):
            _, changed = carry
            return changed

        def fix_body(carry):
            s, _ = carry
            active = 1.0 - s
            t = lax.dot_general(active, m, (((1,), (0,)), ((), ())),
                                preferred_element_type=jnp.float32)
            s_new = jnp.maximum(s0, jnp.where(t > 0.0, 1.0, 0.0))
            return s_new, jnp.any(s_new != s)

        s_fin, _ = lax.while_loop(fix_cond, fix_body, (s0, True))
        supr[ci] = s_fin
        keep_row = 1.0 - s_fin

        def cross(cj, _):
            ox1 = x1r[cj]
            oy1 = y1r[cj]
            ox2 = x2r[cj]
            oy2 = y2r[cj]
            oa = ar[cj]
            d = jnp.where(
                iou_ge(cx1, cy1, cx2, cy2, ca, ox1, oy1, ox2, oy2, oa),
                1.0, 0.0)
            t = lax.dot_general(keep_row, d, (((1,), (0,)), ((), ())),
                                preferred_element_type=jnp.float32)
            supr[cj] = jnp.maximum(supr[cj], jnp.where(t > 0.0, 1.0, 0.0))
            return 0

        lax.fori_loop(ci + 1, nb, cross, 0, unroll=False)
        return 0

    lax.fori_loop(0, nb, chunk_step, 0, unroll=False)


def _run_nms_sorted(x1s, y1s, x2s, y2s, areas_s, chunk):
    n = x1s.shape[0]
    c = chunk
    nb = -(-n // c)
    npad = nb * c
    pad = npad - n

    def prep(v):
        return jnp.pad(v, (0, pad)).reshape(nb, 1, c)

    x1r = prep(x1s)
    y1r = prep(y1s)
    x2r = prep(x2s)
    y2r = prep(y2s)
    arr = prep(areas_s)
    sup0 = jnp.pad(jnp.zeros((n,), jnp.float32), (0, pad),
                   constant_values=1.0).reshape(nb, 1, c)

    sup = pl.pallas_call(
        functools.partial(_nms_chunk_kernel, nb, c),
        out_shape=jax.ShapeDtypeStruct((nb, 1, c), jnp.float32),
    )(x1r, y1r, x2r, y2r, arr, sup0)
    return sup.reshape(npad)[:n]


def kernel(boxes, scores):
    n = boxes.shape[0]
    x1 = boxes[:, 0]
    y1 = boxes[:, 1]
    x2 = boxes[:, 2]
    y2 = boxes[:, 3]
    areas = (x2 - x1 + 1.0) * (y2 - y1 + 1.0)
    order = jnp.argsort(-scores)

    x1s = x1[order]
    y1s = y1[order]
    x2s = x2[order]
    y2s = y2[order]
    areas_s = areas[order]

    sup = _run_nms_sorted(x1s, y1s, x2s, y2s, areas_s, chunk=1024)

    keep_sorted = sup < 0.5
    keep = jnp.zeros((n,), dtype=bool).at[order].set(keep_sorted)
    keep_f = keep.astype(boxes.dtype)
    labels = jnp.zeros((n, 1), dtype=boxes.dtype)
    bbox_pred = jnp.concatenate([labels, scores[:, None], boxes],
                                axis=1) * keep_f[:, None]
    bbox_num = jnp.sum(keep).astype(jnp.int32)[None]
    nms_keep_idx = jnp.nonzero(keep, size=n, fill_value=0)[0]
    return bbox_pred, bbox_num, nms_keep_idx


# speedup vs baseline: 203.0123x; 1.0017x over previous
import functools

import jax
import jax.numpy as jnp
from jax import lax
from jax.experimental import pallas as pl
from jax.experimental.pallas import tpu as pltpu

_THRESH = 0.6


def _nms_chunk_kernel(nb, c,
                      x1r, y1r, x2r, y2r, ar,
                      sup0, supr):
    supr[...] = sup0[...]

    row_i = lax.broadcasted_iota(jnp.int32, (c, c), 0)
    col_i = lax.broadcasted_iota(jnp.int32, (c, c), 1)
    tri = (col_i > row_i)

    def to_col(v):
        return jnp.transpose(v, (1, 0))

    def iou_ge(cx1, cy1, cx2, cy2, ca, rx1, ry1, rx2, ry2, ra):
        xx1 = jnp.maximum(cx1, rx1)
        yy1 = jnp.maximum(cy1, ry1)
        xx2 = jnp.minimum(cx2, rx2)
        yy2 = jnp.minimum(cy2, ry2)
        w = jnp.maximum(0.0, xx2 - xx1 + 1.0)
        h = jnp.maximum(0.0, yy2 - yy1 + 1.0)
        inter = w * h
        union = ca + ra - inter
        return (inter / union) >= _THRESH

    def chunk_step(ci, _):
        rx1 = x1r[ci]
        ry1 = y1r[ci]
        rx2 = x2r[ci]
        ry2 = y2r[ci]
        ra = ar[ci]

        cx1 = to_col(rx1)
        cy1 = to_col(ry1)
        cx2 = to_col(rx2)
        cy2 = to_col(ry2)
        ca = to_col(ra)

        hit = iou_ge(cx1, cy1, cx2, cy2, ca, rx1, ry1, rx2, ry2, ra)
        m = jnp.where(hit & tri, 1.0, 0.0)

        s0 = supr[ci]

        def fix_cond(carry):
            _, changed = carry
            return changed

        def fix_body(carry):
            s, _ = carry
            active = 1.0 - s
            t = lax.dot_general(active, m, (((1,), (0,)), ((), ())),
                                preferred_element_type=jnp.float32)
            s_new = jnp.maximum(s0, jnp.where(t > 0.0, 1.0, 0.0))
            return s_new, jnp.any(s_new != s)

        s_fin, _ = lax.while_loop(fix_cond, fix_body, (s0, True))
        supr[ci] = s_fin
        keep_row = 1.0 - s_fin

        def cross(cj, _):
            ox1 = x1r[cj]
            oy1 = y1r[cj]
            ox2 = x2r[cj]
            oy2 = y2r[cj]
            oa = ar[cj]
            d = jnp.where(
                iou_ge(cx1, cy1, cx2, cy2, ca, ox1, oy1, ox2, oy2, oa),
                1.0, 0.0)
            t = lax.dot_general(keep_row, d, (((1,), (0,)), ((), ())),
                                preferred_element_type=jnp.float32)
            supr[cj] = jnp.maximum(supr[cj], jnp.where(t > 0.0, 1.0, 0.0))
            return 0

        lax.fori_loop(ci + 1, nb, cross, 0, unroll=False)
        return 0

    lax.fori_loop(0, nb, chunk_step, 0, unroll=False)


def _run_nms_sorted(x1s, y1s, x2s, y2s, areas_s, chunk):
    n = x1s.shape[0]
    c = chunk
    nb = -(-n // c)
    npad = nb * c
    pad = npad - n

    def prep(v):
        return jnp.pad(v, (0, pad)).reshape(nb, 1, c)

    x1r = prep(x1s)
    y1r = prep(y1s)
    x2r = prep(x2s)
    y2r = prep(y2s)
    arr = prep(areas_s)
    sup0 = jnp.pad(jnp.zeros((n,), jnp.float32), (0, pad),
                   constant_values=1.0).reshape(nb, 1, c)

    sup = pl.pallas_call(
        functools.partial(_nms_chunk_kernel, nb, c),
        out_shape=jax.ShapeDtypeStruct((nb, 1, c), jnp.float32),
    )(x1r, y1r, x2r, y2r, arr, sup0)
    return sup.reshape(npad)[:n]


def kernel(boxes, scores):
    n = boxes.shape[0]
    x1 = boxes[:, 0]
    y1 = boxes[:, 1]
    x2 = boxes[:, 2]
    y2 = boxes[:, 3]
    areas = (x2 - x1 + 1.0) * (y2 - y1 + 1.0)
    order = jnp.argsort(-scores)

    x1s = x1[order]
    y1s = y1[order]
    x2s = x2[order]
    y2s = y2[order]
    areas_s = areas[order]

    sup = _run_nms_sorted(x1s, y1s, x2s, y2s, areas_s, chunk=2048)

    keep_sorted = sup < 0.5
    keep = jnp.zeros((n,), dtype=bool).at[order].set(keep_sorted)
    keep_f = keep.astype(boxes.dtype)
    labels = jnp.zeros((n, 1), dtype=boxes.dtype)
    bbox_pred = jnp.concatenate([labels, scores[:, None], boxes],
                                axis=1) * keep_f[:, None]
    bbox_num = jnp.sum(keep).astype(jnp.int32)[None]
    nms_keep_idx = jnp.nonzero(keep, size=n, fill_value=0)[0]
    return bbox_pred, bbox_num, nms_keep_idx
